# Initial kernel scaffold; baseline (speedup 1.0000x reference)
import jax
import jax.numpy as jnp
from jax.experimental import pallas as pl


def kernel(x_internal, x_external, edge_index_txn, edge_index_wd, edge_index_dep, Wl1_txn, Wr1_txn, b1_txn, Wl1_wd, Wr1_wd, b1_wd, Wl1_dep, Wr1_dep, b1_dep, Wl2_txn, Wr2_txn, b2_txn, Wl2_wd, Wr2_wd, b2_wd, Wl2_dep, Wr2_dep, b2_dep, W_out, b_out):
    raise NotImplementedError("write your pallas kernel here")


# speedup vs baseline: 2.3624x; 2.3624x over previous
import functools

import jax
import jax.numpy as jnp
from jax import lax
from jax.experimental import pallas as pl
from jax.experimental.pallas import tpu as pltpu
from jax.experimental.pallas import tpu_sc as plsc

N = 50000
D = 128
DC = 4
DW = 32
NP = 50016
GARB = 50000
NGARB = 8
NC = 2
NS = 16
NW = NC * NS
EB = 128
K = 6
SB = K * EB
RPT = NP // NS
BN = 2000
_TC_PARAMS = pltpu.CompilerParams(vmem_limit_bytes=100 * 1024 * 1024)


def _make_agg(e_pad: int, with_cnt: bool):
  nseg = DC + (1 if with_cnt else 0)
  t_per = e_pad // NW
  grps = t_per // SB
  assert t_per % SB == 0
  mesh = plsc.VectorSubcoreMesh(core_axis_name="c", subcore_axis_name="s")

  @functools.partial(
      pl.kernel,
      out_type=jax.ShapeDtypeStruct((NC, nseg, NP, DW), jnp.float32),
      mesh=mesh,
      scratch_types=[
          pltpu.VMEM((SB,), jnp.int32),
          pltpu.VMEM((SB,), jnp.int32),
          pltpu.VMEM((SB, DW), jnp.float32),
          pltpu.VMEM_SHARED((NP, DW), jnp.float32),
          pltpu.SemaphoreType.DMA,
      ],
      compiler_params=pltpu.CompilerParams(use_tc_tiling_on_sc=False),
  )
  def agg(x0, x1, x2, x3, src_hbm, dst_hbm, ones_hbm, zeros_hbm, out_hbm,
          si_v, di_v, rv_v, acc_sh, gsem):
    cid = lax.axis_index("c")
    sid = lax.axis_index("s")
    ebase = (sid * NC + cid) * t_per
    row0 = sid * RPT
    tables = [x0, x1, x2, x3]

    for seg in range(nseg):
      pltpu.sync_copy(zeros_hbm, acc_sh.at[pl.ds(row0, RPT)])
      plsc.subcore_barrier()

      if seg < DC:
        table = tables[seg]

        def body(g, c):
          pltpu.sync_copy(src_hbm.at[pl.ds(ebase + g * SB, SB)], si_v)
          pltpu.sync_copy(dst_hbm.at[pl.ds(ebase + g * SB, SB)], di_v)
          pltpu.async_copy(table.at[si_v], rv_v, gsem).wait()
          pltpu.sync_copy(rv_v, acc_sh.at[di_v], add=True)
          return c
        lax.fori_loop(0, grps, body, 0)
      else:
        for j in range(K):
          pltpu.sync_copy(ones_hbm, rv_v.at[pl.ds(j * EB, EB)])

        def cbody(g, c):
          pltpu.sync_copy(dst_hbm.at[pl.ds(ebase + g * SB, SB)], di_v)
          pltpu.sync_copy(rv_v, acc_sh.at[di_v], add=True)
          return c
        lax.fori_loop(0, grps, cbody, 0)

      plsc.subcore_barrier()
      pltpu.sync_copy(acc_sh.at[pl.ds(row0, RPT)],
                      out_hbm.at[cid, seg, pl.ds(row0, RPT)])
      plsc.subcore_barrier()

  return agg


def _dense_two_rel(acc_a, acc_b, x_dmaj, wl_a, wr_a, b_a, wl_b, wr_b, b_b):
  def body(aa_ref, ab_ref, x_ref, wla_ref, wra_ref, ba_ref,
           wlb_ref, wrb_ref, bb_ref, out_ref):
    aa = aa_ref[0] + aa_ref[1]
    ab = ab_ref[0] + ab_ref[1]
    ra = 1.0 / jnp.maximum(aa[DC, :, 0], 1.0)
    rb = 1.0 / jnp.maximum(ab[DC, :, 0], 1.0)
    agg_a = jnp.concatenate([aa[d] for d in range(DC)], axis=1) * ra[:, None]
    agg_b = jnp.concatenate([ab[d] for d in range(DC)], axis=1) * rb[:, None]
    x = jnp.concatenate([x_ref[d] for d in range(DC)], axis=1)
    res = (jnp.dot(agg_a, wla_ref[...], preferred_element_type=jnp.float32)
           + jnp.dot(agg_b, wlb_ref[...], preferred_element_type=jnp.float32)
           + jnp.dot(x, wra_ref[...] + wrb_ref[...],
                     preferred_element_type=jnp.float32)
           + ba_ref[...] + bb_ref[...]) * 0.5
    h = jnp.tanh(res)
    for d in range(DC):
      out_ref[d] = h[:, d * DW:(d + 1) * DW]

  grid = (N // BN,)
  acc_spec = pl.BlockSpec((NC, DC + 1, BN, DW), lambda i: (0, 0, i, 0))
  x_spec = pl.BlockSpec((DC, BN, DW), lambda i: (0, i, 0))
  w_spec = pl.BlockSpec((D, D), lambda i: (0, 0))
  b_spec = pl.BlockSpec((1, D), lambda i: (0, 0))
  return pl.pallas_call(
      body,
      grid=grid,
      in_specs=[acc_spec, acc_spec, x_spec,
                w_spec, w_spec, b_spec, w_spec, w_spec, b_spec],
      out_specs=pl.BlockSpec((DC, BN, DW), lambda i: (0, i, 0)),
      out_shape=jax.ShapeDtypeStruct((DC, N, DW), jnp.float32),
      compiler_params=_TC_PARAMS,
  )(acc_a, acc_b, x_dmaj, wl_a, wr_a, b_a, wl_b, wr_b, b_b)


def _dense_one_rel(acc, x_dmaj, wl, wr, b):
  def body(a_ref, x_ref, wl_ref, wr_ref, b_ref, out_ref):
    a = a_ref[0] + a_ref[1]
    r = 1.0 / jnp.maximum(a[DC, :, 0], 1.0)
    agg = jnp.concatenate([a[d] for d in range(DC)], axis=1) * r[:, None]
    x = jnp.concatenate([x_ref[d] for d in range(DC)], axis=1)
    res = (jnp.dot(agg, wl_ref[...], preferred_element_type=jnp.float32)
           + jnp.dot(x, wr_ref[...], preferred_element_type=jnp.float32)
           + b_ref[...])
    h = jnp.tanh(res)
    for d in range(DC):
      out_ref[d] = h[:, d * DW:(d + 1) * DW]

  grid = (N // BN,)
  acc_spec = pl.BlockSpec((NC, DC + 1, BN, DW), lambda i: (0, 0, i, 0))
  x_spec = pl.BlockSpec((DC, BN, DW), lambda i: (0, i, 0))
  w_spec = pl.BlockSpec((D, D), lambda i: (0, 0))
  b_spec = pl.BlockSpec((1, D), lambda i: (0, 0))
  return pl.pallas_call(
      body,
      grid=grid,
      in_specs=[acc_spec, x_spec, w_spec, w_spec, b_spec],
      out_specs=pl.BlockSpec((DC, BN, DW), lambda i: (0, i, 0)),
      out_shape=jax.ShapeDtypeStruct((DC, N, DW), jnp.float32),
      compiler_params=_TC_PARAMS,
  )(acc, x_dmaj, wl, wr, b)


def _dense_final(acc_a, acc_b, cnt_a, cnt_b, x_dmaj,
                 wl_a, wr_a, b_a, wl_b, wr_b, b_b, wout_pad, bout_pad):
  def body(aa_ref, ab_ref, ca_ref, cb_ref, x_ref, wla_ref, wra_ref, ba_ref,
           wlb_ref, wrb_ref, bb_ref, wo_ref, bo_ref, out_ref):
    aa = aa_ref[0] + aa_ref[1]
    ab = ab_ref[0] + ab_ref[1]
    ra = 1.0 / jnp.maximum(ca_ref[0, 0, :, 0] + ca_ref[1, 0, :, 0], 1.0)
    rb = 1.0 / jnp.maximum(cb_ref[0, 0, :, 0] + cb_ref[1, 0, :, 0], 1.0)
    agg_a = jnp.concatenate([aa[d] for d in range(DC)], axis=1) * ra[:, None]
    agg_b = jnp.concatenate([ab[d] for d in range(DC)], axis=1) * rb[:, None]
    x = jnp.concatenate([x_ref[d] for d in range(DC)], axis=1)
    res = (jnp.dot(agg_a, wla_ref[...], preferred_element_type=jnp.float32)
           + jnp.dot(agg_b, wlb_ref[...], preferred_element_type=jnp.float32)
           + jnp.dot(x, wra_ref[...] + wrb_ref[...],
                     preferred_element_type=jnp.float32)
           + ba_ref[...] + bb_ref[...]) * 0.5
    h = jnp.tanh(res)
    logits = jnp.dot(h, wo_ref[...],
                     preferred_element_type=jnp.float32) + bo_ref[...]
    l0 = logits[:, 0:1]
    l1 = logits[:, 1:2]
    m = jnp.maximum(l0, l1)
    e0 = jnp.exp(l0 - m)
    e1 = jnp.exp(l1 - m)
    s = e0 + e1
    out_ref[...] = jnp.concatenate(
        [e0 / s, e1 / s, jnp.zeros((BN, D - 2), jnp.float32)], axis=1)

  grid = (N // BN,)
  acc4_spec = pl.BlockSpec((NC, DC, BN, DW), lambda i: (0, 0, i, 0))
  cnt_spec = pl.BlockSpec((NC, 1, BN, DW), lambda i: (0, DC, i, 0))
  x_spec = pl.BlockSpec((DC, BN, DW), lambda i: (0, i, 0))
  w_spec = pl.BlockSpec((D, D), lambda i: (0, 0))
  b_spec = pl.BlockSpec((1, D), lambda i: (0, 0))
  return pl.pallas_call(
      body,
      grid=grid,
      in_specs=[acc4_spec, acc4_spec, cnt_spec, cnt_spec, x_spec,
                w_spec, w_spec, b_spec, w_spec, w_spec, b_spec,
                w_spec, b_spec],
      out_specs=pl.BlockSpec((BN, D), lambda i: (i, 0)),
      out_shape=jax.ShapeDtypeStruct((N, D), jnp.float32),
      compiler_params=_TC_PARAMS,
  )(acc_a, acc_b, cnt_a, cnt_b, x_dmaj,
    wl_a, wr_a, b_a, wl_b, wr_b, b_b, wout_pad, bout_pad)


def _pad_edges(ei, e_pad):
  e = ei.shape[1]
  pad = jnp.arange(e_pad - e, dtype=jnp.int32)
  src = jnp.concatenate([ei[0], pad % N])
  dst = jnp.concatenate([ei[1], GARB + pad % NGARB])
  return src, dst


def _dmaj(x):
  return jnp.transpose(x.reshape(N, DC, DW), (1, 0, 2))


def kernel(x_internal, x_external, edge_index_txn, edge_index_wd,
           edge_index_dep,
           Wl1_txn, Wr1_txn, b1_txn, Wl1_wd, Wr1_wd, b1_wd,
           Wl1_dep, Wr1_dep, b1_dep,
           Wl2_txn, Wr2_txn, b2_txn, Wl2_wd, Wr2_wd, b2_wd,
           Wl2_dep, Wr2_dep, b2_dep,
           W_out, b_out):
  e_t = 417792
  e_d = 122880

  src_t, dst_t = _pad_edges(edge_index_txn, e_t)
  src_d, dst_d = _pad_edges(edge_index_dep, e_d)
  src_w, dst_w = _pad_edges(edge_index_wd, e_d)

  xi0 = _dmaj(x_internal)
  xe0 = _dmaj(x_external)
  ones = jnp.ones((EB, DW), jnp.float32)
  zeros = jnp.zeros((RPT, DW), jnp.float32)

  agg_t5 = _make_agg(e_t, with_cnt=True)
  agg_d5 = _make_agg(e_d, with_cnt=True)
  agg_t4 = _make_agg(e_t, with_cnt=False)
  agg_d4 = _make_agg(e_d, with_cnt=False)

  b1t = b1_txn.reshape(1, D)
  b1d = b1_dep.reshape(1, D)
  b1w = b1_wd.reshape(1, D)
  b2t = b2_txn.reshape(1, D)
  b2d = b2_dep.reshape(1, D)
  wout_pad = jnp.zeros((D, D), jnp.float32).at[:, :2].set(W_out)
  bout_pad = jnp.zeros((1, D), jnp.float32).at[0, :2].set(b_out)

  acc_t1 = agg_t5(xi0[0], xi0[1], xi0[2], xi0[3], src_t, dst_t, ones, zeros)
  acc_d1 = agg_d5(xe0[0], xe0[1], xe0[2], xe0[3], src_d, dst_d, ones, zeros)
  acc_w1 = agg_d5(xi0[0], xi0[1], xi0[2], xi0[3], src_w, dst_w, ones, zeros)

  xi1 = _dense_two_rel(acc_t1, acc_d1, xi0,
                       Wl1_txn, Wr1_txn, b1t, Wl1_dep, Wr1_dep, b1d)
  xe1 = _dense_one_rel(acc_w1, xe0, Wl1_wd, Wr1_wd, b1w)

  acc_t2 = agg_t4(xi1[0], xi1[1], xi1[2], xi1[3], src_t, dst_t, ones, zeros)
  acc_d2 = agg_d4(xe1[0], xe1[1], xe1[2], xe1[3], src_d, dst_d, ones, zeros)

  full = _dense_final(acc_t2, acc_d2, acc_t1, acc_d1, xi1,
                      Wl2_txn, Wr2_txn, b2t, Wl2_dep, Wr2_dep, b2d,
                      wout_pad, bout_pad)
  return full[:, :2]
